# Initial kernel scaffold; baseline (speedup 1.0000x reference)
import functools

import jax
import jax.numpy as jnp
from jax import lax
from jax.experimental import pallas as pl
from jax.experimental.pallas import tpu as pltpu
from jax.experimental.pallas import tpu_sc as plsc

_N = 10000
_E = 320000
_H = 64
_OUT = 3
_MP = 5
_CW = 125
_NW = 32
_EPW = _E // _NW
_CPW = _EPW // _CW
_BE = 4000
_NPT = _N // 16

_F32 = jnp.float32


def _ln(t, g, b):
    mu = jnp.mean(t, axis=-1, keepdims=True)
    var = jnp.mean((t - mu) ** 2, axis=-1, keepdims=True)
    return (t - mu) / jnp.sqrt(var + 1e-5) * g + b


def _dot(a, b):
    return jnp.dot(a, b, preferred_element_type=_F32)


def _full(a):
    nd = a.ndim
    return pl.BlockSpec(a.shape, lambda i, _n=nd: (0,) * _n)


def _edge_enc_body(ef, w1, b1, w2, b2, w3, b3, g, bb, out):
    t = jnp.maximum(_dot(ef[...], w1[...]) + b1[...], 0.0)
    t = jnp.maximum(_dot(t, w2[...]) + b2[...], 0.0)
    t = _dot(t, w3[...]) + b3[...]
    out[...] = _ln(t, g[...], bb[...])


def _edge_encode(ef, p):
    w1, w2, w3 = p['W']
    b1, b2, b3 = [b.reshape(1, -1) for b in p['b']]
    g = p['ln_g'].reshape(1, -1)
    bb = p['ln_b'].reshape(1, -1)
    args = (w1, b1, w2, b2, w3, b3, g, bb)
    return pl.pallas_call(
        _edge_enc_body,
        grid=(_E // _BE,),
        in_specs=[pl.BlockSpec((_BE, 16), lambda i: (i, 0))] + [_full(a) for a in args],
        out_specs=pl.BlockSpec((_BE, _H), lambda i: (i, 0)),
        out_shape=jax.ShapeDtypeStruct((_E, _H), _F32),
    )(ef, *args)


def _edge_mlp_body(h, e, w1e, b1, w2, b2, w3, b3, g, bb, newe, enext):
    ev = e[...]
    t = jnp.maximum(h[...] + _dot(ev, w1e[...]) + b1[...], 0.0)
    t = jnp.maximum(_dot(t, w2[...]) + b2[...], 0.0)
    t = _dot(t, w3[...]) + b3[...]
    ne = _ln(t, g[...], bb[...])
    newe[...] = ne
    enext[...] = ev + ne


def _edge_mlp(h, e, p):
    w1 = p['W'][0]
    w1e = w1[2 * _H:, :]
    b1, b2, b3 = [b.reshape(1, -1) for b in p['b']]
    w2, w3 = p['W'][1], p['W'][2]
    g = p['ln_g'].reshape(1, -1)
    bb = p['ln_b'].reshape(1, -1)
    args = (w1e, b1, w2, b2, w3, b3, g, bb)
    return pl.pallas_call(
        _edge_mlp_body,
        grid=(_E // _BE,),
        in_specs=[pl.BlockSpec((_BE, _H), lambda i: (i, 0)),
                  pl.BlockSpec((_BE, _H), lambda i: (i, 0))] + [_full(a) for a in args],
        out_specs=[pl.BlockSpec((_BE, _H), lambda i: (i, 0)),
                   pl.BlockSpec((_BE, _H), lambda i: (i, 0))],
        out_shape=[jax.ShapeDtypeStruct((_E, _H), _F32),
                   jax.ShapeDtypeStruct((_E, _H), _F32)],
    )(h, e, *args)


def _node_enc_body(nf, w1, b1, w2, b2, w3, b3, g, bb, ws, wr, xo, xso, xro):
    t = jnp.maximum(_dot(nf[...], w1[...]) + b1[...], 0.0)
    t = jnp.maximum(_dot(t, w2[...]) + b2[...], 0.0)
    t = _dot(t, w3[...]) + b3[...]
    x = _ln(t, g[...], bb[...])
    xo[...] = x
    xso[...] = _dot(x, ws[...])
    xro[...] = _dot(x, wr[...])


def _node_encode(nf, p, ws, wr):
    w1, w2, w3 = p['W']
    b1, b2, b3 = [b.reshape(1, -1) for b in p['b']]
    g = p['ln_g'].reshape(1, -1)
    bb = p['ln_b'].reshape(1, -1)
    sh = jax.ShapeDtypeStruct((_N, _H), _F32)
    return pl.pallas_call(
        _node_enc_body,
        out_shape=[sh, sh, sh],
    )(nf, w1, b1, w2, b2, w3, b3, g, bb, ws, wr)


def _node_mlp_body(x, parts, wnx, wna, b1, w2, b2, w3, b3, g, bb, ws, wr,
                   xo, xso, xro):
    xv = x[...]
    agg = parts[0] + parts[1]
    t = jnp.maximum(_dot(xv, wnx[...]) + _dot(agg, wna[...]) + b1[...], 0.0)
    t = jnp.maximum(_dot(t, w2[...]) + b2[...], 0.0)
    t = _dot(t, w3[...]) + b3[...]
    xn = xv + _ln(t, g[...], bb[...])
    xo[...] = xn
    xso[...] = _dot(xn, ws[...])
    xro[...] = _dot(xn, wr[...])


def _node_mlp(x, parts, p, ws, wr):
    w1 = p['W'][0]
    wnx = w1[:_H, :]
    wna = w1[_H:, :]
    b1, b2, b3 = [b.reshape(1, -1) for b in p['b']]
    w2, w3 = p['W'][1], p['W'][2]
    g = p['ln_g'].reshape(1, -1)
    bb = p['ln_b'].reshape(1, -1)
    sh = jax.ShapeDtypeStruct((_N, _H), _F32)
    return pl.pallas_call(
        _node_mlp_body,
        out_shape=[sh, sh, sh],
    )(x, parts, wnx, wna, b1, w2, b2, w3, b3, g, bb, ws, wr)


def _decode_body(x, w1, b1, w2, b2, w3, b3, out):
    t = jnp.maximum(_dot(x[...], w1[...]) + b1[...], 0.0)
    t = jnp.maximum(_dot(t, w2[...]) + b2[...], 0.0)
    out[...] = _dot(t, w3[...]) + b3[...]


def _decode(x, p):
    w1, w2, w3 = p['W']
    b1, b2, b3 = [b.reshape(1, -1) for b in p['b']]
    return pl.pallas_call(
        _decode_body,
        out_shape=jax.ShapeDtypeStruct((_N, _OUT), _F32),
    )(x, w1, b1, w2, b2, w3, b3)


_mesh = plsc.VectorSubcoreMesh(core_axis_name="c", subcore_axis_name="s")


@functools.partial(
    pl.kernel,
    out_type=jax.ShapeDtypeStruct((_E, _H), _F32),
    mesh=_mesh,
    scratch_types=[
        pltpu.VMEM((_CPW, _CW), jnp.int32),
        pltpu.VMEM((_CPW, _CW), jnp.int32),
        pltpu.VMEM((_CW, _H), _F32),
        pltpu.SemaphoreType.DMA,
    ],
)
def _sc_gather_add(xs_hbm, xr_hbm, s2_hbm, r2_hbm, out_hbm, sidx, ridx, rows, sem):
    w = lax.axis_index("c") * 16 + lax.axis_index("s")
    pltpu.sync_copy(s2_hbm.at[pl.ds(w * _CPW, _CPW)], sidx)
    pltpu.sync_copy(r2_hbm.at[pl.ds(w * _CPW, _CPW)], ridx)

    def body(j, carry):
        pltpu.async_copy(xs_hbm.at[sidx.at[j]], rows, sem).wait()
        pltpu.async_copy(xr_hbm.at[ridx.at[j]], rows, sem, add=True).wait()
        pltpu.sync_copy(rows, out_hbm.at[pl.ds(w * _EPW + j * _CW, _CW)])
        return carry

    lax.fori_loop(0, _CPW, body, 0)


@functools.partial(
    pl.kernel,
    out_type=jax.ShapeDtypeStruct((2, _N, _H), _F32),
    mesh=_mesh,
    scratch_types=[
        pltpu.VMEM((_CPW, _CW), jnp.int32),
        pltpu.VMEM((_CW, _H), _F32),
        pltpu.VMEM_SHARED((_N, _H), _F32),
    ],
)
def _sc_scatter_add(newe_hbm, r2_hbm, zeros_hbm, out_hbm, ridx, rows, aggsh):
    c = lax.axis_index("c")
    s = lax.axis_index("s")
    w = c * 16 + s
    pltpu.sync_copy(zeros_hbm, aggsh.at[pl.ds(s * _NPT, _NPT)])
    pltpu.sync_copy(r2_hbm.at[pl.ds(w * _CPW, _CPW)], ridx)
    plsc.subcore_barrier()

    def body(j, carry):
        pltpu.sync_copy(newe_hbm.at[pl.ds(w * _EPW + j * _CW, _CW)], rows)
        pltpu.sync_copy(rows, aggsh.at[ridx.at[j]], add=True)
        return carry

    lax.fori_loop(0, _CPW, body, 0)
    plsc.subcore_barrier()
    pltpu.sync_copy(aggsh.at[pl.ds(s * _NPT, _NPT)],
                    out_hbm.at[c].at[pl.ds(s * _NPT, _NPT)])


def kernel(node_features, edge_features, senders, receivers, params):
    s2 = senders.astype(jnp.int32).reshape(_NW * _CPW, _CW)
    r2 = receivers.astype(jnp.int32).reshape(_NW * _CPW, _CW)
    zeros = jnp.zeros((_NPT, _H), _F32)

    blocks = params['blocks']
    w1_0 = blocks[0]['edge']['W'][0]
    e = _edge_encode(edge_features, params['enc_edge'])
    x, xs, xr = _node_encode(node_features, params['enc_node'],
                             w1_0[:_H, :], w1_0[_H:2 * _H, :])

    for k in range(_MP):
        blk = blocks[k]
        h = _sc_gather_add(xs, xr, s2, r2)
        new_e, e = _edge_mlp(h, e, blk['edge'])
        parts = _sc_scatter_add(new_e, r2, zeros)
        if k + 1 < _MP:
            w1n = blocks[k + 1]['edge']['W'][0]
            wsn, wrn = w1n[:_H, :], w1n[_H:2 * _H, :]
        else:
            wsn, wrn = w1_0[:_H, :], w1_0[_H:2 * _H, :]
        x, xs, xr = _node_mlp(x, parts, blk['node'], wsn, wrn)

    return _decode(x, params['dec'])


# speedup vs baseline: 2.3575x; 2.3575x over previous
import functools

import jax
import jax.numpy as jnp
from jax import lax
from jax.experimental import pallas as pl
from jax.experimental.pallas import tpu as pltpu
from jax.experimental.pallas import tpu_sc as plsc

_N = 10000
_E = 320000
_H = 64
_OUT = 3
_MP = 5
_CW = 80
_NW = 32
_EPW = _E // _NW
_CPW = _EPW // _CW
_BE = 4000
_NPAD = 10240
_NPT = _NPAD // 16

_F32 = jnp.float32


def _ln(t, g, b):
    mu = jnp.mean(t, axis=-1, keepdims=True)
    var = jnp.mean((t - mu) ** 2, axis=-1, keepdims=True)
    return (t - mu) / jnp.sqrt(var + 1e-5) * g + b


def _dot(a, b):
    return jnp.dot(a, b, preferred_element_type=_F32)


def _full(a):
    nd = a.ndim
    return pl.BlockSpec(a.shape, lambda i, _n=nd: (0,) * _n)


def _edge_enc_body(ef, w1, b1, w2, b2, w3, b3, g, bb, out):
    t = jnp.maximum(_dot(ef[...], w1[...]) + b1[...], 0.0)
    t = jnp.maximum(_dot(t, w2[...]) + b2[...], 0.0)
    t = _dot(t, w3[...]) + b3[...]
    out[...] = _ln(t, g[...], bb[...])


def _edge_encode(ef, p):
    w1, w2, w3 = p['W']
    b1, b2, b3 = [b.reshape(1, -1) for b in p['b']]
    g = p['ln_g'].reshape(1, -1)
    bb = p['ln_b'].reshape(1, -1)
    args = (w1, b1, w2, b2, w3, b3, g, bb)
    return pl.pallas_call(
        _edge_enc_body,
        grid=(_E // _BE,),
        in_specs=[pl.BlockSpec((_BE, 16), lambda i: (i, 0))] + [_full(a) for a in args],
        out_specs=pl.BlockSpec((_BE, _H), lambda i: (i, 0)),
        out_shape=jax.ShapeDtypeStruct((_E, _H), _F32),
    )(ef, *args)


def _edge_mlp_body(h, e, w1e, b1, w2, b2, w3, b3, g, bb, newe, enext):
    ev = e[...]
    t = jnp.maximum(h[...] + _dot(ev, w1e[...]) + b1[...], 0.0)
    t = jnp.maximum(_dot(t, w2[...]) + b2[...], 0.0)
    t = _dot(t, w3[...]) + b3[...]
    ne = _ln(t, g[...], bb[...])
    newe[...] = ne
    enext[...] = ev + ne


def _edge_mlp(h, e, p):
    w1 = p['W'][0]
    w1e = w1[2 * _H:, :]
    b1, b2, b3 = [b.reshape(1, -1) for b in p['b']]
    w2, w3 = p['W'][1], p['W'][2]
    g = p['ln_g'].reshape(1, -1)
    bb = p['ln_b'].reshape(1, -1)
    args = (w1e, b1, w2, b2, w3, b3, g, bb)
    return pl.pallas_call(
        _edge_mlp_body,
        grid=(_E // _BE,),
        in_specs=[pl.BlockSpec((_BE, _H), lambda i: (i, 0)),
                  pl.BlockSpec((_BE, _H), lambda i: (i, 0))] + [_full(a) for a in args],
        out_specs=[pl.BlockSpec((_BE, _H), lambda i: (i, 0)),
                   pl.BlockSpec((_BE, _H), lambda i: (i, 0))],
        out_shape=[jax.ShapeDtypeStruct((_E, _H), _F32),
                   jax.ShapeDtypeStruct((_E, _H), _F32)],
    )(h, e, *args)


def _node_enc_body(nf, w1, b1, w2, b2, w3, b3, g, bb, ws, wr, xo, xso, xro):
    t = jnp.maximum(_dot(nf[...], w1[...]) + b1[...], 0.0)
    t = jnp.maximum(_dot(t, w2[...]) + b2[...], 0.0)
    t = _dot(t, w3[...]) + b3[...]
    x = _ln(t, g[...], bb[...])
    xo[...] = x
    xso[...] = _dot(x, ws[...])
    xro[...] = _dot(x, wr[...])


def _node_encode(nf, p, ws, wr):
    w1, w2, w3 = p['W']
    b1, b2, b3 = [b.reshape(1, -1) for b in p['b']]
    g = p['ln_g'].reshape(1, -1)
    bb = p['ln_b'].reshape(1, -1)
    sh = jax.ShapeDtypeStruct((_N, _H), _F32)
    return pl.pallas_call(
        _node_enc_body,
        out_shape=[sh, sh, sh],
    )(nf, w1, b1, w2, b2, w3, b3, g, bb, ws, wr)


def _node_mlp_body(x, parts, wnx, wna, b1, w2, b2, w3, b3, g, bb, ws, wr,
                   xo, xso, xro):
    xv = x[...]
    pv = parts[...]
    agg = pv[0, :_N] + pv[1, :_N]
    t = jnp.maximum(_dot(xv, wnx[...]) + _dot(agg, wna[...]) + b1[...], 0.0)
    t = jnp.maximum(_dot(t, w2[...]) + b2[...], 0.0)
    t = _dot(t, w3[...]) + b3[...]
    xn = xv + _ln(t, g[...], bb[...])
    xo[...] = xn
    xso[...] = _dot(xn, ws[...])
    xro[...] = _dot(xn, wr[...])


def _node_mlp(x, parts, p, ws, wr):
    w1 = p['W'][0]
    wnx = w1[:_H, :]
    wna = w1[_H:, :]
    b1, b2, b3 = [b.reshape(1, -1) for b in p['b']]
    w2, w3 = p['W'][1], p['W'][2]
    g = p['ln_g'].reshape(1, -1)
    bb = p['ln_b'].reshape(1, -1)
    sh = jax.ShapeDtypeStruct((_N, _H), _F32)
    return pl.pallas_call(
        _node_mlp_body,
        out_shape=[sh, sh, sh],
    )(x, parts, wnx, wna, b1, w2, b2, w3, b3, g, bb, ws, wr)


def _decode_body(x, w1, b1, w2, b2, w3, b3, out):
    t = jnp.maximum(_dot(x[...], w1[...]) + b1[...], 0.0)
    t = jnp.maximum(_dot(t, w2[...]) + b2[...], 0.0)
    out[...] = _dot(t, w3[...]) + b3[...]


def _decode(x, p):
    w1, w2, w3 = p['W']
    b1, b2, b3 = [b.reshape(1, -1) for b in p['b']]
    return pl.pallas_call(
        _decode_body,
        out_shape=jax.ShapeDtypeStruct((_N, _OUT), _F32),
    )(x, w1, b1, w2, b2, w3, b3)


_mesh = plsc.VectorSubcoreMesh(core_axis_name="c", subcore_axis_name="s")


@functools.partial(
    pl.kernel,
    out_type=jax.ShapeDtypeStruct((_E, _H), _F32),
    mesh=_mesh,
    compiler_params=pltpu.CompilerParams(use_tc_tiling_on_sc=False),
    scratch_types=[
        pltpu.VMEM((_CPW, _CW), jnp.int32),
        pltpu.VMEM((_CPW, _CW), jnp.int32),
        pltpu.VMEM((_CW, _H), _F32),
        pltpu.SemaphoreType.DMA,
    ],
)
def _sc_gather_add(xs_hbm, xr_hbm, s2_hbm, r2_hbm, out_hbm, sidx, ridx, rows, sem):
    w = lax.axis_index("c") * 16 + lax.axis_index("s")
    pltpu.sync_copy(s2_hbm.at[w], sidx)
    pltpu.sync_copy(r2_hbm.at[w], ridx)

    def body(j, carry):
        pltpu.async_copy(xs_hbm.at[sidx.at[j]], rows, sem).wait()
        pltpu.async_copy(xr_hbm.at[ridx.at[j]], rows, sem, add=True).wait()
        pltpu.sync_copy(rows, out_hbm.at[pl.ds(w * _EPW + j * _CW, _CW)])
        return carry

    lax.fori_loop(0, _CPW, body, 0)


@functools.partial(
    pl.kernel,
    out_type=jax.ShapeDtypeStruct((2, _NPAD, _H), _F32),
    mesh=_mesh,
    compiler_params=pltpu.CompilerParams(use_tc_tiling_on_sc=False),
    scratch_types=[
        pltpu.VMEM((_CPW, _CW), jnp.int32),
        pltpu.VMEM((_CW, _H), _F32),
        pltpu.VMEM_SHARED((_NPAD, _H), _F32),
    ],
)
def _sc_scatter_add(newe_hbm, r2_hbm, zeros_hbm, out_hbm, ridx, rows, aggsh):
    c = lax.axis_index("c")
    s = lax.axis_index("s")
    w = c * 16 + s
    pltpu.sync_copy(zeros_hbm, aggsh.at[pl.ds(s * _NPT, _NPT)])
    pltpu.sync_copy(r2_hbm.at[w], ridx)
    plsc.subcore_barrier()

    def body(j, carry):
        pltpu.sync_copy(newe_hbm.at[pl.ds(w * _EPW + j * _CW, _CW)], rows)
        pltpu.sync_copy(rows, aggsh.at[ridx.at[j]], add=True)
        return carry

    lax.fori_loop(0, _CPW, body, 0)
    plsc.subcore_barrier()
    pltpu.sync_copy(aggsh.at[pl.ds(s * _NPT, _NPT)],
                    out_hbm.at[c].at[pl.ds(s * _NPT, _NPT)])


def kernel(node_features, edge_features, senders, receivers, params):
    s2 = senders.astype(jnp.int32).reshape(_NW, _CPW, _CW)
    r2 = receivers.astype(jnp.int32).reshape(_NW, _CPW, _CW)
    zeros = jnp.zeros((_NPT, _H), _F32)

    blocks = params['blocks']
    w1_0 = blocks[0]['edge']['W'][0]
    e = _edge_encode(edge_features, params['enc_edge'])
    x, xs, xr = _node_encode(node_features, params['enc_node'],
                             w1_0[:_H, :], w1_0[_H:2 * _H, :])

    for k in range(_MP):
        blk = blocks[k]
        h = _sc_gather_add(xs, xr, s2, r2)
        new_e, e = _edge_mlp(h, e, blk['edge'])
        parts = _sc_scatter_add(new_e, r2, zeros)
        if k + 1 < _MP:
            w1n = blocks[k + 1]['edge']['W'][0]
            wsn, wrn = w1n[:_H, :], w1n[_H:2 * _H, :]
        else:
            wsn, wrn = w1_0[:_H, :], w1_0[_H:2 * _H, :]
        x, xs, xr = _node_mlp(x, parts, blk['node'], wsn, wrn)

    return _decode(x, params['dec'])
